# baseline (speedup 1.0000x reference)
import functools

import jax
import jax.numpy as jnp
import numpy as np
from jax import lax
from jax.experimental import pallas as pl
from jax.experimental.pallas import tpu as pltpu
from jax.experimental.pallas import tpu_sc as plsc

_GRID = 24
_BW = 2
_TOKENS = 577
_BH = 48
_NT = 4
_LANES = 16
_NW = 32


def _border_cols() -> np.ndarray:
    cols = []
    for r in range(_GRID):
        for c in range(_GRID):
            if r < _BW or r >= _GRID - _BW or c < _BW or c >= _GRID - _BW:
                cols.append(1 + r * _GRID + c)
    return np.asarray(sorted(cols), dtype=np.int32)


_COLS = _border_cols()
_NIDX = _COLS.size
_SCALE = np.float32(0.1 / (_NT * _BH * _NIDX))


@functools.partial(
    pl.kernel,
    out_type=jax.ShapeDtypeStruct((_NW, _LANES), jnp.float32),
    mesh=plsc.VectorSubcoreMesh(core_axis_name="c", subcore_axis_name="s"),
    compiler_params=pltpu.CompilerParams(needs_layout_passes=False),
    scratch_types=(
        [pltpu.VMEM((_NIDX,), jnp.int32)]
        + [pltpu.VMEM((_TOKENS,), jnp.float32) for _ in range(2 * _NT)]
        + [pltpu.VMEM((_LANES,), jnp.float32), pltpu.SemaphoreType.DMA]
    ),
)
def _border_sums(a0, a1, a2, a3, cols_hbm, out_hbm,
                 cols_v, b0, b1, b2, b3, b4, b5, b6, b7, acc_v, sem):
    arrs = (a0, a1, a2, a3)
    bufs_lo = (b0, b1, b2, b3)
    bufs_hi = (b4, b5, b6, b7)
    wid = lax.axis_index("s") * 2 + lax.axis_index("c")
    row_lo = wid
    row_hi = jnp.minimum(wid + _NW, _BH - 1)

    copies = []
    for a, b in zip(arrs, bufs_lo):
        copies.append(pltpu.async_copy(a.at[row_lo, 0], b, sem))
    for a, b in zip(arrs, bufs_hi):
        copies.append(pltpu.async_copy(a.at[row_hi, 0], b, sem))

    pltpu.sync_copy(cols_hbm, cols_v)
    idx = [cols_v[pl.ds(j * _LANES, _LANES)] for j in range(_NIDX // _LANES)]

    def row_sum(buf):
        s = plsc.load_gather(buf, [idx[0]])
        for j in range(1, len(idx)):
            s = s + plsc.load_gather(buf, [idx[j]])
        return s

    for cp in copies[:_NT]:
        cp.wait()
    total = row_sum(b0) + row_sum(b1) + row_sum(b2) + row_sum(b3)
    for cp in copies[_NT:]:
        cp.wait()
    hi = row_sum(b4) + row_sum(b5) + row_sum(b6) + row_sum(b7)
    gate = jnp.where(wid + _NW < _BH, jnp.float32(1.0), jnp.float32(0.0))
    total = total + hi * gate
    acc_v[...] = total
    pltpu.sync_copy(acc_v, out_hbm.at[wid])


def _finish(p_ref, o_ref):
    o_ref[0, 0] = jnp.sum(p_ref[...]) * _SCALE


def kernel(attn_0, attn_1, attn_2, attn_3):
    flat = [jnp.reshape(a, (_BH, _TOKENS, _TOKENS))
            for a in (attn_0, attn_1, attn_2, attn_3)]
    parts = _border_sums(*flat, jnp.asarray(_COLS))
    total = pl.pallas_call(
        _finish,
        out_shape=jax.ShapeDtypeStruct((1, 1), jnp.float32),
        out_specs=pl.BlockSpec(memory_space=pltpu.SMEM),
    )(parts)
    return total[0, 0]


# speedup vs baseline: 6.5444x; 6.5444x over previous
import functools

import jax
import jax.numpy as jnp
import numpy as np
from jax import lax
from jax.experimental import pallas as pl
from jax.experimental.pallas import tpu as pltpu
from jax.experimental.pallas import tpu_sc as plsc

_GRID = 24
_BW = 2
_TOKENS = 577
_BH = 48
_NT = 4
_LANES = 16
_NW = 32
_ROWS = _NT * _BH
_RPW = _ROWS // _NW


def _border_cols() -> np.ndarray:
    cols = []
    for r in range(_GRID):
        for c in range(_GRID):
            if r < _BW or r >= _GRID - _BW or c < _BW or c >= _GRID - _BW:
                cols.append(1 + r * _GRID + c)
    return np.asarray(sorted(cols), dtype=np.int32)


_COLS = _border_cols()
_NIDX = _COLS.size
_SCALE = np.float32(0.1 / (_NT * _BH * _NIDX))


def _extract(a0, a1, a2, a3, o_ref):
    for t, ref in enumerate((a0, a1, a2, a3)):
        o_ref[pl.ds(t * _BH, _BH)] = jnp.reshape(
            ref[:, :, 0, :], (_BH, _TOKENS))


@functools.partial(
    pl.kernel,
    out_type=jax.ShapeDtypeStruct((_NW, _LANES), jnp.float32),
    mesh=plsc.VectorSubcoreMesh(core_axis_name="c", subcore_axis_name="s"),
    compiler_params=pltpu.CompilerParams(needs_layout_passes=False),
    scratch_types=(
        [pltpu.VMEM((_NIDX,), jnp.int32)]
        + [pltpu.VMEM((_TOKENS,), jnp.float32) for _ in range(_RPW)]
        + [pltpu.VMEM((_LANES,), jnp.float32), pltpu.SemaphoreType.DMA]
    ),
)
def _border_sums(rows_hbm, cols_hbm, out_hbm,
                 cols_v, b0, b1, b2, b3, b4, b5, acc_v, sem):
    bufs = (b0, b1, b2, b3, b4, b5)
    wid = lax.axis_index("s") * 2 + lax.axis_index("c")
    base = wid * _RPW
    copies = [
        pltpu.async_copy(rows_hbm.at[base + r], b, sem)
        for r, b in enumerate(bufs)
    ]

    pltpu.sync_copy(cols_hbm, cols_v)
    idx = [cols_v[pl.ds(j * _LANES, _LANES)] for j in range(_NIDX // _LANES)]

    def row_sum(buf):
        s = plsc.load_gather(buf, [idx[0]])
        for j in range(1, len(idx)):
            s = s + plsc.load_gather(buf, [idx[j]])
        return s

    total = None
    for cp, b in zip(copies, bufs):
        cp.wait()
        s = row_sum(b)
        total = s if total is None else total + s
    acc_v[...] = total
    pltpu.sync_copy(acc_v, out_hbm.at[wid])


def _finish(p_ref, o_ref):
    o_ref[0, 0] = jnp.sum(p_ref[...]) * _SCALE


def kernel(attn_0, attn_1, attn_2, attn_3):
    shape = attn_0.shape
    spec = pl.BlockSpec(
        (shape[0], shape[1], 8, _TOKENS), lambda i: (0, 0, 0, 0))
    rows = pl.pallas_call(
        _extract,
        grid=(1,),
        out_shape=jax.ShapeDtypeStruct((_ROWS, _TOKENS), jnp.float32),
        in_specs=[spec] * _NT,
        out_specs=pl.BlockSpec((_ROWS, _TOKENS), lambda i: (0, 0)),
    )(attn_0, attn_1, attn_2, attn_3)
    parts = _border_sums(rows, jnp.asarray(_COLS))
    total = pl.pallas_call(
        _finish,
        out_shape=jax.ShapeDtypeStruct((1, 1), jnp.float32),
        out_specs=pl.BlockSpec(memory_space=pltpu.SMEM),
    )(parts)
    return total[0, 0]


# speedup vs baseline: 53.7670x; 8.2158x over previous
import functools

import jax
import jax.numpy as jnp
import numpy as np
from jax import lax
from jax.experimental import pallas as pl
from jax.experimental.pallas import tpu as pltpu
from jax.experimental.pallas import tpu_sc as plsc

_GRID = 24
_BW = 2
_TOKENS = 577
_BH = 48
_NT = 4
_LANES = 16
_NW = 32
_ROWS = _NT * _BH
_RPW = _ROWS // _NW


def _border_cols() -> np.ndarray:
    cols = []
    for r in range(_GRID):
        for c in range(_GRID):
            if r < _BW or r >= _GRID - _BW or c < _BW or c >= _GRID - _BW:
                cols.append(1 + r * _GRID + c)
    return np.asarray(sorted(cols), dtype=np.int32)


_COLS = _border_cols()
_NIDX = _COLS.size
_SCALE = np.float32(0.1 / (_NT * _BH * _NIDX))


def _extract(a0, a1, a2, a3, o_ref):
    for t, ref in enumerate((a0, a1, a2, a3)):
        o_ref[pl.ds(t * _BH, _BH)] = jnp.reshape(
            ref[:, :, 0, :], (_BH, _TOKENS))


@functools.partial(
    pl.kernel,
    out_type=jax.ShapeDtypeStruct((_NW, _LANES), jnp.float32),
    mesh=plsc.VectorSubcoreMesh(core_axis_name="c", subcore_axis_name="s"),
    compiler_params=pltpu.CompilerParams(needs_layout_passes=False),
    scratch_types=(
        [pltpu.VMEM((_NIDX,), jnp.int32)]
        + [pltpu.VMEM((_TOKENS,), jnp.float32) for _ in range(_RPW)]
        + [pltpu.VMEM((_LANES,), jnp.float32), pltpu.SemaphoreType.DMA]
    ),
)
def _border_sums(rows_hbm, cols_hbm, out_hbm,
                 cols_v, b0, b1, b2, b3, b4, b5, acc_v, sem):
    bufs = (b0, b1, b2, b3, b4, b5)
    wid = lax.axis_index("s") * 2 + lax.axis_index("c")
    base = wid * _RPW
    copies = [
        pltpu.async_copy(rows_hbm.at[base + r], b, sem)
        for r, b in enumerate(bufs)
    ]

    pltpu.sync_copy(cols_hbm, cols_v)
    idx = [cols_v[pl.ds(j * _LANES, _LANES)] for j in range(_NIDX // _LANES)]

    def row_sum(buf):
        s = plsc.load_gather(buf, [idx[0]])
        for j in range(1, len(idx)):
            s = s + plsc.load_gather(buf, [idx[j]])
        return s

    total = None
    for cp, b in zip(copies, bufs):
        cp.wait()
        s = row_sum(b)
        total = s if total is None else total + s
    acc_v[...] = total
    pltpu.sync_copy(acc_v, out_hbm.at[wid])


def _finish(p_ref, o_ref):
    o_ref[0, 0] = jnp.sum(p_ref[...]) * _SCALE


def kernel(attn_0, attn_1, attn_2, attn_3):
    rows = jnp.reshape(
        jnp.stack([a[:, :, 0, :] for a in (attn_0, attn_1, attn_2, attn_3)]),
        (_ROWS, _TOKENS))
    parts = _border_sums(rows, jnp.asarray(_COLS))
    total = pl.pallas_call(
        _finish,
        out_shape=jax.ShapeDtypeStruct((1, 1), jnp.float32),
        out_specs=pl.BlockSpec(memory_space=pltpu.SMEM),
    )(parts)
    return total[0, 0]


# speedup vs baseline: 262.1710x; 4.8761x over previous
import jax
import jax.numpy as jnp
import numpy as np
from jax.experimental import pallas as pl
from jax.experimental.pallas import tpu as pltpu

_GRID = 24
_BW = 2
_TOKENS = 577
_BH = 48
_NT = 4
_ROWS = _NT * _BH


def _border_cols() -> np.ndarray:
    cols = []
    for r in range(_GRID):
        for c in range(_GRID):
            if r < _BW or r >= _GRID - _BW or c < _BW or c >= _GRID - _BW:
                cols.append(1 + r * _GRID + c)
    return np.asarray(sorted(cols), dtype=np.int32)


_COLS = _border_cols()
_NIDX = _COLS.size
_SCALE = np.float32(0.1 / (_NT * _BH * _NIDX))
_MASK = np.zeros((1, _TOKENS), dtype=np.float32)
_MASK[0, _COLS] = 1.0


def _border_mean(r_ref, m_ref, o_ref):
    x = r_ref[...]
    o_ref[0, 0] = jnp.sum(x * m_ref[...]) * _SCALE


def kernel(attn_0, attn_1, attn_2, attn_3):
    rows = jnp.reshape(
        jnp.stack([a[:, :, 0, :] for a in (attn_0, attn_1, attn_2, attn_3)]),
        (_ROWS, _TOKENS))
    total = pl.pallas_call(
        _border_mean,
        out_shape=jax.ShapeDtypeStruct((1, 1), jnp.float32),
        out_specs=pl.BlockSpec(memory_space=pltpu.SMEM),
    )(rows, jnp.asarray(_MASK))
    return total[0, 0]
